# baseline (speedup 1.0000x reference)
import functools

import jax
import jax.numpy as jnp
from jax import lax
from jax.experimental import pallas as pl
from jax.experimental.pallas import tpu as pltpu
from jax.experimental.pallas import tpu_sc as plsc

N = 10000
E = 160000
D = 256
HD = 128
NH = N // 2
NC = 2
NS = 16
L = 16
E_PAD = 163840
EPT = E_PAD // NS
ECH = 64
NCH = EPT // ECH
DEG_CH = 1024
DRN = 312
DRL = 320
NW = NC * NS

NP = 10240
NPT = NP // NS

_vsm = plsc.VectorSubcoreMesh(
    core_axis_name="c", subcore_axis_name="s", num_cores=NC, num_subcores=NS)


def _nrsqrt(x):
    xi = plsc.bitcast(x, jnp.int32)
    y = plsc.bitcast(jnp.int32(0x5F3759DF) - (xi >> 1), jnp.float32)
    for _ in range(3):
        y = y * (1.5 - 0.5 * x * y * y)
    return y


@functools.partial(
    pl.kernel,
    out_type=(jax.ShapeDtypeStruct((NP,), jnp.float32),
              jax.ShapeDtypeStruct((NC, NS, EPT), jnp.int32),
              jax.ShapeDtypeStruct((NC, NS, EPT), jnp.int32),
              jax.ShapeDtypeStruct((NC, NS, EPT), jnp.float32),
              jax.ShapeDtypeStruct((NW, L), jnp.int32)),
    mesh=_vsm,
    scratch_types=[
        pltpu.VMEM((NP,), jnp.float32),
        pltpu.VMEM((DEG_CH,), jnp.int32),
        pltpu.VMEM((DEG_CH,), jnp.int32),
        pltpu.VMEM((DEG_CH,), jnp.float32),
        pltpu.VMEM((NS, NPT), jnp.float32),
        pltpu.VMEM((NPT,), jnp.float32),
        pltpu.VMEM((NP,), jnp.float32),
        pltpu.VMEM((EPT + 2 * L,), jnp.int32),
        pltpu.VMEM((EPT + 2 * L,), jnp.int32),
        pltpu.VMEM((EPT + 2 * L,), jnp.float32),
        pltpu.VMEM((L,), jnp.int32),
        pltpu.VMEM_SHARED((NS, NP), jnp.float32),
        pltpu.VMEM_SHARED((NP,), jnp.float32),
    ],
    compiler_params=pltpu.CompilerParams(needs_layout_passes=False),
)
def _prep(src_hbm, dst_hbm, ew_hbm, dinv_hbm, fsrc_hbm, fdst_hbm, fse_hbm,
          cnt_hbm,
          deg_loc, ebuf_s, ebuf_d, ebuf_w, d2buf, dinv_loc, dinv_all,
          fs_loc, fd_loc, fe_loc, cnt_v, deg_sh, dinv_sh):
    c = lax.axis_index("c")
    s = lax.axis_index("s")
    zeros = jnp.zeros((L,), jnp.float32)
    izeros = jnp.zeros((L,), jnp.int32)

    def z_body(i, _):
        deg_loc[pl.ds(i * L, L)] = zeros
        return 0
    lax.fori_loop(0, NP // L, z_body, 0)

    def zf_body(i, _):
        fs_loc[pl.ds(i * L, L)] = izeros
        fd_loc[pl.ds(i * L, L)] = izeros
        fe_loc[pl.ds(i * L, L)] = zeros
        return 0
    lax.fori_loop(0, EPT // L + 2, zf_body, 0)

    def deg_chunk(ci, _):
        base = s * EPT + ci * DEG_CH
        pltpu.sync_copy(dst_hbm.at[pl.ds(base, DEG_CH)], ebuf_d)
        pltpu.sync_copy(ew_hbm.at[pl.ds(base, DEG_CH)], ebuf_w)

        def inner(i, _):
            dv = ebuf_d[pl.ds(i * L, L)]
            wv = ebuf_w[pl.ds(i * L, L)]
            plsc.addupdate_scatter(deg_loc, [dv], wv)
            return 0
        lax.fori_loop(0, DEG_CH // L, inner, 0)
        return 0
    lax.fori_loop(0, EPT // DEG_CH, deg_chunk, 0)

    pltpu.sync_copy(deg_loc, deg_sh.at[s])
    plsc.subcore_barrier()
    pltpu.sync_copy(deg_sh.at[:, pl.ds(s * NPT, NPT)], d2buf)

    def red_body(i, _):
        acc = d2buf[0, pl.ds(i * L, L)]
        for r in range(1, NS):
            acc = acc + d2buf[r, pl.ds(i * L, L)]
        dinv_loc[pl.ds(i * L, L)] = _nrsqrt(acc + 1.0)
        return 0
    lax.fori_loop(0, NPT // L, red_body, 0)
    pltpu.sync_copy(dinv_loc, dinv_sh.at[pl.ds(s * NPT, NPT)])

    @pl.when(c == 0)
    def _():
        pltpu.sync_copy(dinv_loc, dinv_hbm.at[pl.ds(s * NPT, NPT)])
    plsc.subcore_barrier()
    pltpu.sync_copy(dinv_sh, dinv_all)

    lo = c * NH

    def flt_chunk(ci, cnt):
        base = s * EPT + ci * DEG_CH
        pltpu.sync_copy(src_hbm.at[pl.ds(base, DEG_CH)], ebuf_s)
        pltpu.sync_copy(dst_hbm.at[pl.ds(base, DEG_CH)], ebuf_d)
        pltpu.sync_copy(ew_hbm.at[pl.ds(base, DEG_CH)], ebuf_w)

        def inner(i, cnt):
            sv = ebuf_s[pl.ds(i * L, L)]
            dv = ebuf_d[pl.ds(i * L, L)]
            wv = ebuf_w[pl.ds(i * L, L)]
            dl = dv - lo
            m = (dl >= 0) & (dl < NH) & (wv != 0.0)
            se = wv * plsc.load_gather(dinv_all, [sv])
            plsc.store_compressed(fs_loc.at[pl.ds(cnt, L)], sv, mask=m)
            plsc.store_compressed(fd_loc.at[pl.ds(cnt, L)], dl, mask=m)
            plsc.store_compressed(fe_loc.at[pl.ds(cnt, L)], se, mask=m)
            return cnt + jnp.sum(m.astype(jnp.int32))
        return lax.fori_loop(0, DEG_CH // L, inner, cnt)
    cnt = lax.fori_loop(0, EPT // DEG_CH, flt_chunk, jnp.int32(0))

    def tail_z(i, _):
        fs_loc[pl.ds(cnt + i * L, L)] = izeros
        fd_loc[pl.ds(cnt + i * L, L)] = izeros
        fe_loc[pl.ds(cnt + i * L, L)] = zeros
        return 0
    lax.fori_loop(0, 2, tail_z, 0)

    w = c * NS + s
    pltpu.sync_copy(fs_loc.at[pl.ds(0, EPT)], fsrc_hbm.at[c, s])
    pltpu.sync_copy(fd_loc.at[pl.ds(0, EPT)], fdst_hbm.at[c, s])
    pltpu.sync_copy(fe_loc.at[pl.ds(0, EPT)], fse_hbm.at[c, s])
    cnt_v[...] = jnp.full((L,), cnt, jnp.int32)
    pltpu.sync_copy(cnt_v, cnt_hbm.at[w])


@functools.partial(
    pl.kernel,
    out_type=jax.ShapeDtypeStruct((N, D), jnp.float32),
    mesh=_vsm,
    scratch_types=[
        pltpu.VMEM((2, ECH, HD), jnp.int32),
        pltpu.VMEM((2, ECH, D), jnp.float32),
        pltpu.VMEM((3, ECH), jnp.int32),
        pltpu.VMEM((3, ECH), jnp.int32),
        pltpu.VMEM((3, ECH), jnp.float32),
        pltpu.VMEM((ECH,), jnp.float32),
        pltpu.VMEM((L,), jnp.int32),
        pltpu.SemaphoreType.DMA,
        pltpu.SemaphoreType.DMA,
        pltpu.SemaphoreType.DMA,
        pltpu.VMEM_SHARED((NH, D), jnp.float32),
    ],
    compiler_params=pltpu.CompilerParams(
        needs_layout_passes=False, use_tc_tiling_on_sc=False),
)
def _agg(h_hbm, src_hbm, dst_hbm, se_hbm, cnt_hbm, z_hbm, o_hbm,
         gbuf2, msg2, src3b, dst3b, se3b, se_buf, cbuf,
         gsem, ssem, isem, acc_sh):
    c = lax.axis_index("c")
    s = lax.axis_index("s")
    w = c * NS + s

    @pl.when(s < NS - 1)
    def _():
        pltpu.sync_copy(z_hbm.at[pl.ds(s * DRN, DRN)],
                        acc_sh.at[pl.ds(s * DRN, DRN)])

    @pl.when(s == NS - 1)
    def _():
        pltpu.sync_copy(z_hbm.at[pl.ds((NS - 1) * DRN, DRL)],
                        acc_sh.at[pl.ds((NS - 1) * DRN, DRL)])

    pltpu.sync_copy(cnt_hbm.at[w], cbuf)
    cnt = cbuf[pl.ds(0, L)][0]
    tc = jnp.maximum((cnt + (ECH - 1)) // ECH, 1)
    plsc.subcore_barrier()

    def start_idx(j):
        r = lax.rem(j, 3)
        pltpu.async_copy(src_hbm.at[c, s, pl.ds(j * ECH, ECH)],
                         src3b.at[r], isem)
        pltpu.async_copy(dst_hbm.at[c, s, pl.ds(j * ECH, ECH)],
                         dst3b.at[r], isem)
        pltpu.async_copy(se_hbm.at[c, s, pl.ds(j * ECH, ECH)],
                         se3b.at[r], isem)

    def drain_idx():
        pltpu.make_async_copy(
            src_hbm.at[c, s, pl.ds(0, ECH)], src3b.at[0], isem).wait()
        pltpu.make_async_copy(
            dst_hbm.at[c, s, pl.ds(0, ECH)], dst3b.at[0], isem).wait()
        pltpu.make_async_copy(
            se_hbm.at[c, s, pl.ds(0, ECH)], se3b.at[0], isem).wait()

    def drain_gather():
        pltpu.make_async_copy(
            h_hbm.at[pl.ds(0, ECH)], gbuf2.at[0], gsem).wait()

    def drain_scatter():
        pltpu.make_async_copy(
            z_hbm.at[pl.ds(0, ECH)], msg2.at[0], ssem).wait()

    start_idx(0)

    @pl.when(tc > 1)
    def _():
        start_idx(1)
    drain_idx()
    pltpu.async_copy(h_hbm.at[src3b.at[0]], gbuf2.at[0], gsem)

    def chunk(j, _):
        jb = lax.rem(j, 2)
        nb = lax.rem(j + 1, 2)
        jr = lax.rem(j, 3)

        @pl.when(j + 1 < tc)
        def _():
            drain_idx()

        @pl.when(j + 2 < tc)
        def _():
            start_idx(j + 2)

        @pl.when(j + 1 < tc)
        def _():
            pltpu.async_copy(
                h_hbm.at[src3b.at[lax.rem(j + 1, 3)]], gbuf2.at[nb], gsem)

        drain_gather()

        @pl.when(j >= 2)
        def _():
            drain_scatter()

        for k in range(ECH // L):
            se_buf[pl.ds(k * L, L)] = se3b[jr, pl.ds(k * L, L)]

        jbs = jnp.full((L,), jb, jnp.int32)
        evens = lax.iota(jnp.int32, L) * 2

        def edge(e, _):
            sv = plsc.load_gather(se_buf, [jnp.full((L,), e, jnp.int32)])
            es = jnp.full((L,), e, jnp.int32)
            for k in range(D // 32):
                w16 = gbuf2[jb, e, pl.ds(k * L, L)]
                hb = plsc.bitcast(w16, jnp.bfloat16)
                a, b = plsc.unpack(hb, format=plsc.PackFormat.INTERLEAVED)
                plsc.store_scatter(msg2, [jbs, es, evens + (k * 32)],
                                   a * sv)
                plsc.store_scatter(msg2, [jbs, es, evens + (k * 32 + 1)],
                                   b * sv)
            return 0
        lax.fori_loop(0, ECH, edge, 0)
        pltpu.async_copy(msg2.at[jb], acc_sh.at[dst3b.at[jr]],
                         ssem, add=True)
        return 0
    lax.fori_loop(0, tc, chunk, 0)

    @pl.when(tc >= 2)
    def _():
        drain_scatter()
    drain_scatter()

    plsc.subcore_barrier()

    @pl.when(s < NS - 1)
    def _():
        pltpu.sync_copy(acc_sh.at[pl.ds(s * DRN, DRN)],
                        o_hbm.at[pl.ds(c * NH + s * DRN, DRN)])

    @pl.when(s == NS - 1)
    def _():
        pltpu.sync_copy(acc_sh.at[pl.ds((NS - 1) * DRN, DRL)],
                        o_hbm.at[pl.ds(c * NH + (NS - 1) * DRN, DRL)])


def _mm_body(x_ref, w_ref, o0_ref, o1_ref, b_ref):
    h = jnp.dot(x_ref[...], w_ref[...], preferred_element_type=jnp.float32,
                precision=lax.Precision.HIGHEST)
    o0_ref[...] = h[:, :HD]
    o1_ref[...] = h[:, HD:]
    b_ref[...] = h.astype(jnp.bfloat16)


_MMR = 2000


def _matmul_split(x, w):
    return pl.pallas_call(
        _mm_body,
        grid=(N // _MMR,),
        in_specs=[pl.BlockSpec((_MMR, D), lambda i: (i, 0)),
                  pl.BlockSpec((D, D), lambda i: (0, 0))],
        out_specs=[pl.BlockSpec((_MMR, HD), lambda i: (i, 0)),
                   pl.BlockSpec((_MMR, HD), lambda i: (i, 0)),
                   pl.BlockSpec((_MMR, D), lambda i: (i, 0))],
        out_shape=[jax.ShapeDtypeStruct((N, HD), jnp.float32),
                   jax.ShapeDtypeStruct((N, HD), jnp.float32),
                   jax.ShapeDtypeStruct((N, D), jnp.bfloat16)],
    )(x, w)


def _epi_body(a_ref, h0_ref, h1_ref, dv_ref, b_ref, o_ref):
    dv = dv_ref[...]
    dv2 = dv * dv
    b = b_ref[...]
    a = a_ref[...]
    m0 = dv * a[:, :HD] + dv2 * h0_ref[...] + b[:, :HD]
    m1 = dv * a[:, HD:] + dv2 * h1_ref[...] + b[:, HD:]
    o_ref[:, :HD] = jnp.maximum(m0, 0.0)
    o_ref[:, HD:] = jnp.maximum(m1, 0.0)


def _epilogue(a, h0, h1, dinv_col, b_row):
    return pl.pallas_call(
        _epi_body,
        grid=(N // _MMR,),
        in_specs=[pl.BlockSpec((_MMR, D), lambda i: (i, 0)),
                  pl.BlockSpec((_MMR, HD), lambda i: (i, 0)),
                  pl.BlockSpec((_MMR, HD), lambda i: (i, 0)),
                  pl.BlockSpec((_MMR, 1), lambda i: (i, 0)),
                  pl.BlockSpec((1, D), lambda i: (0, 0))],
        out_specs=pl.BlockSpec((_MMR, D), lambda i: (i, 0)),
        out_shape=jax.ShapeDtypeStruct((N, D), jnp.float32),
    )(a, h0, h1, dinv_col, b_row)


def kernel(X, edge_index, edge_weight, W1, b1, W2, b2):
    src = edge_index[0]
    dst = edge_index[1]
    pad_i = jnp.zeros((E_PAD - E,), jnp.int32)
    srcp = jnp.concatenate([src, pad_i])
    dstp = jnp.concatenate([dst, pad_i])
    ewp = jnp.concatenate([edge_weight, jnp.zeros((E_PAD - E,), jnp.float32)])

    dinv1d, fsrc, fdst, fse, cnts = _prep(srcp, dstp, ewp)
    dinv_col = dinv1d[:N].reshape(N, 1)
    b1r = b1.reshape(1, D)
    b2r = b2.reshape(1, D)
    zrows = jnp.zeros((NH, D), jnp.float32)

    def as_i32(hb):
        return lax.bitcast_convert_type(
            hb.reshape(N, D // 2, 2), jnp.int32)

    h1a, h1b, h1bf = _matmul_split(X, W1)
    a1 = _agg(as_i32(h1bf), fsrc, fdst, fse, cnts, zrows)
    out1 = _epilogue(a1, h1a, h1b, dinv_col, b1r)

    h2a, h2b, h2bf = _matmul_split(out1, W2)
    a2 = _agg(as_i32(h2bf), fsrc, fdst, fse, cnts, zrows)
    return _epilogue(a2, h2a, h2b, dinv_col, b2r)


# speedup vs baseline: 1.0094x; 1.0094x over previous
import functools

import jax
import jax.numpy as jnp
from jax import lax
from jax.experimental import pallas as pl
from jax.experimental.pallas import tpu as pltpu
from jax.experimental.pallas import tpu_sc as plsc

N = 10000
E = 160000
D = 256
HD = 128
NH = N // 2
NC = 2
NS = 16
L = 16
E_PAD = 163840
EPT = E_PAD // NS
ECH = 64
NCH = EPT // ECH
DEG_CH = 1024
DRN = 312
DRL = 320
NW = NC * NS

NP = 10240
NPT = NP // NS

_vsm = plsc.VectorSubcoreMesh(
    core_axis_name="c", subcore_axis_name="s", num_cores=NC, num_subcores=NS)


def _nrsqrt(x):
    xi = plsc.bitcast(x, jnp.int32)
    y = plsc.bitcast(jnp.int32(0x5F3759DF) - (xi >> 1), jnp.float32)
    for _ in range(3):
        y = y * (1.5 - 0.5 * x * y * y)
    return y


@functools.partial(
    pl.kernel,
    out_type=(jax.ShapeDtypeStruct((NP,), jnp.float32),
              jax.ShapeDtypeStruct((NC, NS, EPT), jnp.int32),
              jax.ShapeDtypeStruct((NC, NS, EPT), jnp.int32),
              jax.ShapeDtypeStruct((NC, NS, EPT), jnp.float32),
              jax.ShapeDtypeStruct((NW, L), jnp.int32)),
    mesh=_vsm,
    scratch_types=[
        pltpu.VMEM((NP,), jnp.float32),
        pltpu.VMEM((DEG_CH,), jnp.int32),
        pltpu.VMEM((DEG_CH,), jnp.int32),
        pltpu.VMEM((DEG_CH,), jnp.float32),
        pltpu.VMEM((NS, NPT), jnp.float32),
        pltpu.VMEM((NPT,), jnp.float32),
        pltpu.VMEM((NP,), jnp.float32),
        pltpu.VMEM((EPT + 2 * L,), jnp.int32),
        pltpu.VMEM((EPT + 2 * L,), jnp.int32),
        pltpu.VMEM((EPT + 2 * L,), jnp.float32),
        pltpu.VMEM((L,), jnp.int32),
        pltpu.VMEM_SHARED((NS, NP), jnp.float32),
        pltpu.VMEM_SHARED((NP,), jnp.float32),
    ],
    compiler_params=pltpu.CompilerParams(
        needs_layout_passes=False, use_tc_tiling_on_sc=False),
)
def _prep(src_hbm, dst_hbm, ew_hbm, dinv_hbm, fsrc_hbm, fdst_hbm, fse_hbm,
          cnt_hbm,
          deg_loc, ebuf_s, ebuf_d, ebuf_w, d2buf, dinv_loc, dinv_all,
          fs_loc, fd_loc, fe_loc, cnt_v, deg_sh, dinv_sh):
    c = lax.axis_index("c")
    s = lax.axis_index("s")
    zeros = jnp.zeros((L,), jnp.float32)
    izeros = jnp.zeros((L,), jnp.int32)

    def z_body(i, _):
        deg_loc[pl.ds(i * L, L)] = zeros
        return 0
    lax.fori_loop(0, NP // L, z_body, 0)

    def zf_body(i, _):
        fs_loc[pl.ds(i * L, L)] = izeros
        fd_loc[pl.ds(i * L, L)] = izeros
        fe_loc[pl.ds(i * L, L)] = zeros
        return 0
    lax.fori_loop(0, EPT // L + 2, zf_body, 0)

    def deg_chunk(ci, _):
        base = s * EPT + ci * DEG_CH
        pltpu.sync_copy(dst_hbm.at[pl.ds(base, DEG_CH)], ebuf_d)
        pltpu.sync_copy(ew_hbm.at[pl.ds(base, DEG_CH)], ebuf_w)

        def inner(i, _):
            dv = ebuf_d[pl.ds(i * L, L)]
            wv = ebuf_w[pl.ds(i * L, L)]
            plsc.addupdate_scatter(deg_loc, [dv], wv)
            return 0
        lax.fori_loop(0, DEG_CH // L, inner, 0)
        return 0
    lax.fori_loop(0, EPT // DEG_CH, deg_chunk, 0)

    pltpu.sync_copy(deg_loc, deg_sh.at[s])
    plsc.subcore_barrier()
    pltpu.sync_copy(deg_sh.at[:, pl.ds(s * NPT, NPT)], d2buf)

    def red_body(i, _):
        acc = d2buf[0, pl.ds(i * L, L)]
        for r in range(1, NS):
            acc = acc + d2buf[r, pl.ds(i * L, L)]
        dinv_loc[pl.ds(i * L, L)] = _nrsqrt(acc + 1.0)
        return 0
    lax.fori_loop(0, NPT // L, red_body, 0)
    pltpu.sync_copy(dinv_loc, dinv_sh.at[pl.ds(s * NPT, NPT)])

    @pl.when(c == 0)
    def _():
        pltpu.sync_copy(dinv_loc, dinv_hbm.at[pl.ds(s * NPT, NPT)])
    plsc.subcore_barrier()
    pltpu.sync_copy(dinv_sh, dinv_all)

    lo = c * NH

    def flt_chunk(ci, cnt):
        base = s * EPT + ci * DEG_CH
        pltpu.sync_copy(src_hbm.at[pl.ds(base, DEG_CH)], ebuf_s)
        pltpu.sync_copy(dst_hbm.at[pl.ds(base, DEG_CH)], ebuf_d)
        pltpu.sync_copy(ew_hbm.at[pl.ds(base, DEG_CH)], ebuf_w)

        def inner(i, cnt):
            sv = ebuf_s[pl.ds(i * L, L)]
            dv = ebuf_d[pl.ds(i * L, L)]
            wv = ebuf_w[pl.ds(i * L, L)]
            dl = dv - lo
            m = (dl >= 0) & (dl < NH) & (wv != 0.0)
            se = wv * plsc.load_gather(dinv_all, [sv])
            plsc.store_compressed(fs_loc.at[pl.ds(cnt, L)], sv, mask=m)
            plsc.store_compressed(fd_loc.at[pl.ds(cnt, L)], dl, mask=m)
            plsc.store_compressed(fe_loc.at[pl.ds(cnt, L)], se, mask=m)
            return cnt + jnp.sum(m.astype(jnp.int32))
        return lax.fori_loop(0, DEG_CH // L, inner, cnt)
    cnt = lax.fori_loop(0, EPT // DEG_CH, flt_chunk, jnp.int32(0))

    def tail_z(i, _):
        fs_loc[pl.ds(cnt + i * L, L)] = izeros
        fd_loc[pl.ds(cnt + i * L, L)] = izeros
        fe_loc[pl.ds(cnt + i * L, L)] = zeros
        return 0
    lax.fori_loop(0, 2, tail_z, 0)

    w = c * NS + s
    pltpu.sync_copy(fs_loc.at[pl.ds(0, EPT)], fsrc_hbm.at[c, s])
    pltpu.sync_copy(fd_loc.at[pl.ds(0, EPT)], fdst_hbm.at[c, s])
    pltpu.sync_copy(fe_loc.at[pl.ds(0, EPT)], fse_hbm.at[c, s])
    cnt_v[...] = jnp.full((L,), cnt, jnp.int32)
    pltpu.sync_copy(cnt_v, cnt_hbm.at[w])


@functools.partial(
    pl.kernel,
    out_type=jax.ShapeDtypeStruct((N, D), jnp.float32),
    mesh=_vsm,
    scratch_types=[
        pltpu.VMEM((2, ECH, HD), jnp.int32),
        pltpu.VMEM((2, ECH, D), jnp.float32),
        pltpu.VMEM((3, ECH), jnp.int32),
        pltpu.VMEM((3, ECH), jnp.int32),
        pltpu.VMEM((3, ECH), jnp.float32),
        pltpu.VMEM((ECH,), jnp.float32),
        pltpu.VMEM((L,), jnp.int32),
        pltpu.SemaphoreType.DMA,
        pltpu.SemaphoreType.DMA,
        pltpu.SemaphoreType.DMA,
        pltpu.VMEM_SHARED((NH, D), jnp.float32),
    ],
    compiler_params=pltpu.CompilerParams(
        needs_layout_passes=False, use_tc_tiling_on_sc=False),
)
def _agg(h_hbm, src_hbm, dst_hbm, se_hbm, cnt_hbm, z_hbm, o_hbm,
         gbuf2, msg2, src3b, dst3b, se3b, se_buf, cbuf,
         gsem, ssem, isem, acc_sh):
    c = lax.axis_index("c")
    s = lax.axis_index("s")
    w = c * NS + s

    @pl.when(s < NS - 1)
    def _():
        pltpu.sync_copy(z_hbm.at[pl.ds(s * DRN, DRN)],
                        acc_sh.at[pl.ds(s * DRN, DRN)])

    @pl.when(s == NS - 1)
    def _():
        pltpu.sync_copy(z_hbm.at[pl.ds((NS - 1) * DRN, DRL)],
                        acc_sh.at[pl.ds((NS - 1) * DRN, DRL)])

    pltpu.sync_copy(cnt_hbm.at[w], cbuf)
    cnt = cbuf[pl.ds(0, L)][0]
    tc = jnp.maximum((cnt + (ECH - 1)) // ECH, 1)
    plsc.subcore_barrier()

    def start_idx(j):
        r = lax.rem(j, 3)
        pltpu.async_copy(src_hbm.at[c, s, pl.ds(j * ECH, ECH)],
                         src3b.at[r], isem)
        pltpu.async_copy(dst_hbm.at[c, s, pl.ds(j * ECH, ECH)],
                         dst3b.at[r], isem)
        pltpu.async_copy(se_hbm.at[c, s, pl.ds(j * ECH, ECH)],
                         se3b.at[r], isem)

    def drain_idx():
        pltpu.make_async_copy(
            src_hbm.at[c, s, pl.ds(0, ECH)], src3b.at[0], isem).wait()
        pltpu.make_async_copy(
            dst_hbm.at[c, s, pl.ds(0, ECH)], dst3b.at[0], isem).wait()
        pltpu.make_async_copy(
            se_hbm.at[c, s, pl.ds(0, ECH)], se3b.at[0], isem).wait()

    def drain_gather():
        pltpu.make_async_copy(
            h_hbm.at[pl.ds(0, ECH)], gbuf2.at[0], gsem).wait()

    def drain_scatter():
        pltpu.make_async_copy(
            z_hbm.at[pl.ds(0, ECH)], msg2.at[0], ssem).wait()

    start_idx(0)

    @pl.when(tc > 1)
    def _():
        start_idx(1)
    drain_idx()
    pltpu.async_copy(h_hbm.at[src3b.at[0]], gbuf2.at[0], gsem)

    def chunk(j, _):
        jb = lax.rem(j, 2)
        nb = lax.rem(j + 1, 2)
        jr = lax.rem(j, 3)

        @pl.when(j + 1 < tc)
        def _():
            drain_idx()

        @pl.when(j + 2 < tc)
        def _():
            start_idx(j + 2)

        @pl.when(j + 1 < tc)
        def _():
            pltpu.async_copy(
                h_hbm.at[src3b.at[lax.rem(j + 1, 3)]], gbuf2.at[nb], gsem)

        drain_gather()

        @pl.when(j >= 2)
        def _():
            drain_scatter()

        for k in range(ECH // L):
            se_buf[pl.ds(k * L, L)] = se3b[jr, pl.ds(k * L, L)]

        jbs = jnp.full((L,), jb, jnp.int32)
        evens = lax.iota(jnp.int32, L) * 2

        def edge(e, _):
            sv = plsc.load_gather(se_buf, [jnp.full((L,), e, jnp.int32)])
            es = jnp.full((L,), e, jnp.int32)
            for k in range(D // 32):
                w16 = gbuf2[jb, e, pl.ds(k * L, L)]
                hb = plsc.bitcast(w16, jnp.bfloat16)
                a, b = plsc.unpack(hb, format=plsc.PackFormat.INTERLEAVED)
                plsc.store_scatter(msg2, [jbs, es, evens + (k * 32)],
                                   a * sv)
                plsc.store_scatter(msg2, [jbs, es, evens + (k * 32 + 1)],
                                   b * sv)
            return 0
        lax.fori_loop(0, ECH, edge, 0)
        pltpu.async_copy(msg2.at[jb], acc_sh.at[dst3b.at[jr]],
                         ssem, add=True)
        return 0
    lax.fori_loop(0, tc, chunk, 0)

    @pl.when(tc >= 2)
    def _():
        drain_scatter()
    drain_scatter()

    plsc.subcore_barrier()

    @pl.when(s < NS - 1)
    def _():
        pltpu.sync_copy(acc_sh.at[pl.ds(s * DRN, DRN)],
                        o_hbm.at[pl.ds(c * NH + s * DRN, DRN)])

    @pl.when(s == NS - 1)
    def _():
        pltpu.sync_copy(acc_sh.at[pl.ds((NS - 1) * DRN, DRL)],
                        o_hbm.at[pl.ds(c * NH + (NS - 1) * DRN, DRL)])


def _mm_body(x_ref, w_ref, o0_ref, o1_ref, b_ref):
    h = jnp.dot(x_ref[...], w_ref[...], preferred_element_type=jnp.float32,
                precision=lax.Precision.HIGHEST)
    o0_ref[...] = h[:, :HD]
    o1_ref[...] = h[:, HD:]
    b_ref[...] = h.astype(jnp.bfloat16)


_MMR = 2000


def _matmul_split(x, w):
    return pl.pallas_call(
        _mm_body,
        grid=(N // _MMR,),
        in_specs=[pl.BlockSpec((_MMR, D), lambda i: (i, 0)),
                  pl.BlockSpec((D, D), lambda i: (0, 0))],
        out_specs=[pl.BlockSpec((_MMR, HD), lambda i: (i, 0)),
                   pl.BlockSpec((_MMR, HD), lambda i: (i, 0)),
                   pl.BlockSpec((_MMR, D), lambda i: (i, 0))],
        out_shape=[jax.ShapeDtypeStruct((N, HD), jnp.float32),
                   jax.ShapeDtypeStruct((N, HD), jnp.float32),
                   jax.ShapeDtypeStruct((N, D), jnp.bfloat16)],
    )(x, w)


def _epi_body(a_ref, h0_ref, h1_ref, dv_ref, b_ref, o_ref):
    dv = dv_ref[...]
    dv2 = dv * dv
    b = b_ref[...]
    a = a_ref[...]
    m0 = dv * a[:, :HD] + dv2 * h0_ref[...] + b[:, :HD]
    m1 = dv * a[:, HD:] + dv2 * h1_ref[...] + b[:, HD:]
    o_ref[:, :HD] = jnp.maximum(m0, 0.0)
    o_ref[:, HD:] = jnp.maximum(m1, 0.0)


def _epilogue(a, h0, h1, dinv_col, b_row):
    return pl.pallas_call(
        _epi_body,
        grid=(N // _MMR,),
        in_specs=[pl.BlockSpec((_MMR, D), lambda i: (i, 0)),
                  pl.BlockSpec((_MMR, HD), lambda i: (i, 0)),
                  pl.BlockSpec((_MMR, HD), lambda i: (i, 0)),
                  pl.BlockSpec((_MMR, 1), lambda i: (i, 0)),
                  pl.BlockSpec((1, D), lambda i: (0, 0))],
        out_specs=pl.BlockSpec((_MMR, D), lambda i: (i, 0)),
        out_shape=jax.ShapeDtypeStruct((N, D), jnp.float32),
    )(a, h0, h1, dinv_col, b_row)


def kernel(X, edge_index, edge_weight, W1, b1, W2, b2):
    src = edge_index[0]
    dst = edge_index[1]
    pad_i = jnp.zeros((E_PAD - E,), jnp.int32)
    srcp = jnp.concatenate([src, pad_i])
    dstp = jnp.concatenate([dst, pad_i])
    ewp = jnp.concatenate([edge_weight, jnp.zeros((E_PAD - E,), jnp.float32)])

    dinv1d, fsrc, fdst, fse, cnts = _prep(srcp, dstp, ewp)
    dinv_col = dinv1d[:N].reshape(N, 1)
    b1r = b1.reshape(1, D)
    b2r = b2.reshape(1, D)
    zrows = jnp.zeros((NH, D), jnp.float32)

    def as_i32(hb):
        return lax.bitcast_convert_type(
            hb.reshape(N, D // 2, 2), jnp.int32)

    h1a, h1b, h1bf = _matmul_split(X, W1)
    a1 = _agg(as_i32(h1bf), fsrc, fdst, fse, cnts, zrows)
    out1 = _epilogue(a1, h1a, h1b, dinv_col, b1r)

    h2a, h2b, h2bf = _matmul_split(out1, W2)
    a2 = _agg(as_i32(h2bf), fsrc, fdst, fse, cnts, zrows)
    return _epilogue(a2, h2a, h2b, dinv_col, b2r)


# speedup vs baseline: 1.1144x; 1.1040x over previous
import functools

import jax
import jax.numpy as jnp
from jax import lax
from jax.experimental import pallas as pl
from jax.experimental.pallas import tpu as pltpu
from jax.experimental.pallas import tpu_sc as plsc

N = 10000
E = 160000
D = 256
HD = 128
NC = 2
NS = 16
L = 16
E_PAD = 163840
EPT = E_PAD // NS
ECH = 128
NCH = EPT // ECH
DEG_CH = 1024
ESE = E_PAD // (NC * NS)
DRN = 632
DRL = 520
ZRN = 640
ZRL = 400

_vsm = plsc.VectorSubcoreMesh(
    core_axis_name="c", subcore_axis_name="s", num_cores=NC, num_subcores=NS)


def _nrsqrt(x):
    xi = plsc.bitcast(x, jnp.int32)
    y = plsc.bitcast(jnp.int32(0x5F3759DF) - (xi >> 1), jnp.float32)
    for _ in range(3):
        y = y * (1.5 - 0.5 * x * y * y)
    return y


NP = 10240
NPT = NP // NS


@functools.partial(
    pl.kernel,
    out_type=(jax.ShapeDtypeStruct((NP,), jnp.float32),
              jax.ShapeDtypeStruct((E_PAD,), jnp.float32)),
    mesh=_vsm,
    scratch_types=[
        pltpu.VMEM((NP,), jnp.float32),
        pltpu.VMEM((DEG_CH,), jnp.int32),
        pltpu.VMEM((DEG_CH,), jnp.float32),
        pltpu.VMEM((DEG_CH,), jnp.float32),
        pltpu.VMEM((NS, NPT), jnp.float32),
        pltpu.VMEM((NPT,), jnp.float32),
        pltpu.VMEM((NP,), jnp.float32),
        pltpu.VMEM_SHARED((NS, NP), jnp.float32),
        pltpu.VMEM_SHARED((NP,), jnp.float32),
    ],
    compiler_params=pltpu.CompilerParams(needs_layout_passes=False),
)
def _prep(src_hbm, dst_hbm, ew_hbm, dinv_hbm, se_hbm,
          deg_loc, ebuf_i, ebuf_f, se_buf, d2buf, dinv_loc, dinv_all,
          deg_sh, dinv_sh):
    c = lax.axis_index("c")
    s = lax.axis_index("s")
    zeros = jnp.zeros((L,), jnp.float32)

    def z_body(i, _):
        deg_loc[pl.ds(i * L, L)] = zeros
        return 0
    lax.fori_loop(0, NP // L, z_body, 0)

    def deg_chunk(ci, _):
        base = s * EPT + ci * DEG_CH
        pltpu.sync_copy(dst_hbm.at[pl.ds(base, DEG_CH)], ebuf_i)
        pltpu.sync_copy(ew_hbm.at[pl.ds(base, DEG_CH)], ebuf_f)

        def inner(i, _):
            dv = ebuf_i[pl.ds(i * L, L)]
            wv = ebuf_f[pl.ds(i * L, L)]
            plsc.addupdate_scatter(deg_loc, [dv], wv)
            return 0
        lax.fori_loop(0, DEG_CH // L, inner, 0)
        return 0
    lax.fori_loop(0, EPT // DEG_CH, deg_chunk, 0)

    pltpu.sync_copy(deg_loc, deg_sh.at[s])
    plsc.subcore_barrier()
    pltpu.sync_copy(deg_sh.at[:, pl.ds(s * NPT, NPT)], d2buf)

    def red_body(i, _):
        acc = d2buf[0, pl.ds(i * L, L)]
        for r in range(1, NS):
            acc = acc + d2buf[r, pl.ds(i * L, L)]
        dinv_loc[pl.ds(i * L, L)] = _nrsqrt(acc + 1.0)
        return 0
    lax.fori_loop(0, NPT // L, red_body, 0)
    pltpu.sync_copy(dinv_loc, dinv_sh.at[pl.ds(s * NPT, NPT)])

    @pl.when(c == 0)
    def _():
        pltpu.sync_copy(dinv_loc, dinv_hbm.at[pl.ds(s * NPT, NPT)])
    plsc.subcore_barrier()

    pltpu.sync_copy(dinv_sh, dinv_all)
    w = c * NS + s

    def se_chunk(ci, _):
        base = w * ESE + ci * DEG_CH
        pltpu.sync_copy(src_hbm.at[pl.ds(base, DEG_CH)], ebuf_i)
        pltpu.sync_copy(ew_hbm.at[pl.ds(base, DEG_CH)], ebuf_f)

        def inner(i, _):
            sv = ebuf_i[pl.ds(i * L, L)]
            dvv = plsc.load_gather(dinv_all, [sv])
            se_buf[pl.ds(i * L, L)] = ebuf_f[pl.ds(i * L, L)] * dvv
            return 0
        lax.fori_loop(0, DEG_CH // L, inner, 0)
        pltpu.sync_copy(se_buf, se_hbm.at[pl.ds(base, DEG_CH)])
        return 0
    lax.fori_loop(0, ESE // DEG_CH, se_chunk, 0)


@functools.partial(
    pl.kernel,
    out_type=(jax.ShapeDtypeStruct((N, HD), jnp.float32),
              jax.ShapeDtypeStruct((N, HD), jnp.float32)),
    mesh=_vsm,
    scratch_types=[
        pltpu.VMEM((2, ECH, HD // 2), jnp.int32),
        pltpu.VMEM((2, ECH, HD), jnp.float32),
        pltpu.VMEM((3, ECH), jnp.int32),
        pltpu.VMEM((3, ECH), jnp.int32),
        pltpu.VMEM((3, ECH), jnp.float32),
        pltpu.VMEM((ECH,), jnp.float32),
        pltpu.SemaphoreType.DMA,
        pltpu.SemaphoreType.DMA,
        pltpu.SemaphoreType.DMA,
        pltpu.VMEM_SHARED((N, HD), jnp.float32),
    ],
    compiler_params=pltpu.CompilerParams(
        needs_layout_passes=False, use_tc_tiling_on_sc=False),
)
def _agg(h0, h1, src_hbm, dst_hbm, se_hbm, z_hbm, o0, o1,
         gbuf3, msg2, src3b, dst3b, se3b, se_buf, gsem, ssem, isem, acc_sh):
    c = lax.axis_index("c")
    s = lax.axis_index("s")

    @pl.when(s < NS - 1)
    def _():
        pltpu.sync_copy(z_hbm.at[pl.ds(s * DRN, DRN)],
                        acc_sh.at[pl.ds(s * DRN, DRN)])

    @pl.when(s == NS - 1)
    def _():
        pltpu.sync_copy(z_hbm.at[pl.ds((NS - 1) * DRN, DRL)],
                        acc_sh.at[pl.ds((NS - 1) * DRN, DRL)])
    plsc.subcore_barrier()

    def run_half(h_hbm):
        def start_idx(j):
            r = lax.rem(j, 3)
            pltpu.async_copy(src_hbm.at[s, j], src3b.at[r], isem)
            pltpu.async_copy(dst_hbm.at[s, j], dst3b.at[r], isem)
            pltpu.async_copy(se_hbm.at[s, j], se3b.at[r], isem)

        def drain_idx():
            pltpu.make_async_copy(src_hbm.at[s, 0], src3b.at[0], isem).wait()
            pltpu.make_async_copy(dst_hbm.at[s, 0], dst3b.at[0], isem).wait()
            pltpu.make_async_copy(se_hbm.at[s, 0], se3b.at[0], isem).wait()

        def drain_gather():
            pltpu.make_async_copy(
                h_hbm.at[pl.ds(0, ECH)], gbuf3.at[0], gsem).wait()

        def drain_scatter():
            pltpu.make_async_copy(
                z_hbm.at[pl.ds(0, ECH)], msg2.at[0], ssem).wait()

        start_idx(0)
        start_idx(1)
        drain_idx()
        pltpu.async_copy(h_hbm.at[src3b.at[0]], gbuf3.at[0], gsem)

        def chunk(j, _):
            jb = lax.rem(j, 2)
            nb = lax.rem(j + 1, 2)
            jr = lax.rem(j, 3)

            @pl.when(j + 1 < NCH)
            def _():
                drain_idx()

            @pl.when(j + 2 < NCH)
            def _():
                start_idx(j + 2)

            @pl.when(j + 1 < NCH)
            def _():
                pltpu.async_copy(
                    h_hbm.at[src3b.at[lax.rem(j + 1, 3)]], gbuf3.at[nb], gsem)

            drain_gather()

            @pl.when(j >= 2)
            def _():
                drain_scatter()

            for k in range(ECH // L):
                se_buf[pl.ds(k * L, L)] = se3b[jr, pl.ds(k * L, L)]

            jbs = jnp.full((L,), jb, jnp.int32)
            evens = lax.iota(jnp.int32, L) * 2

            def edge(e, _):
                sv = plsc.load_gather(se_buf, [jnp.full((L,), e, jnp.int32)])
                es = jnp.full((L,), e, jnp.int32)
                for k in range(HD // 32):
                    w16 = gbuf3[jb, e, pl.ds(k * L, L)]
                    hb = plsc.bitcast(w16, jnp.bfloat16)
                    a, b = plsc.unpack(hb, format=plsc.PackFormat.INTERLEAVED)
                    plsc.store_scatter(msg2, [jbs, es, evens + (k * 32)],
                                       a * sv)
                    plsc.store_scatter(msg2, [jbs, es, evens + (k * 32 + 1)],
                                       b * sv)
                return 0
            lax.fori_loop(0, ECH, edge, 0)
            pltpu.async_copy(msg2.at[jb], acc_sh.at[dst3b.at[jr]],
                             ssem, add=True)
            return 0
        lax.fori_loop(0, NCH, chunk, 0)
        drain_scatter()
        drain_scatter()

    @pl.when(c == 0)
    def _():
        run_half(h0)

    @pl.when(c == 1)
    def _():
        run_half(h1)

    plsc.subcore_barrier()

    def drain(o_hbm):
        @pl.when(s < NS - 1)
        def _():
            pltpu.sync_copy(acc_sh.at[pl.ds(s * DRN, DRN)],
                            o_hbm.at[pl.ds(s * DRN, DRN)])

        @pl.when(s == NS - 1)
        def _():
            pltpu.sync_copy(acc_sh.at[pl.ds((NS - 1) * DRN, DRL)],
                            o_hbm.at[pl.ds((NS - 1) * DRN, DRL)])

    @pl.when(c == 0)
    def _():
        drain(o0)

    @pl.when(c == 1)
    def _():
        drain(o1)


def _mm_body(x_ref, w_ref, o0_ref, o1_ref, b0_ref, b1_ref):
    h = jnp.dot(x_ref[...], w_ref[...], preferred_element_type=jnp.float32,
                precision=lax.Precision.HIGHEST)
    h0 = h[:, :HD]
    h1 = h[:, HD:]
    o0_ref[...] = h0
    o1_ref[...] = h1
    b0_ref[...] = h0.astype(jnp.bfloat16)
    b1_ref[...] = h1.astype(jnp.bfloat16)


_MMR = 2000


def _matmul_split(x, w):
    return pl.pallas_call(
        _mm_body,
        grid=(N // _MMR,),
        in_specs=[pl.BlockSpec((_MMR, D), lambda i: (i, 0)),
                  pl.BlockSpec((D, D), lambda i: (0, 0))],
        out_specs=[pl.BlockSpec((_MMR, HD), lambda i: (i, 0)),
                   pl.BlockSpec((_MMR, HD), lambda i: (i, 0)),
                   pl.BlockSpec((_MMR, HD), lambda i: (i, 0)),
                   pl.BlockSpec((_MMR, HD), lambda i: (i, 0))],
        out_shape=[jax.ShapeDtypeStruct((N, HD), jnp.float32),
                   jax.ShapeDtypeStruct((N, HD), jnp.float32),
                   jax.ShapeDtypeStruct((N, HD), jnp.bfloat16),
                   jax.ShapeDtypeStruct((N, HD), jnp.bfloat16)],
    )(x, w)


def _epi_body(a0_ref, a1_ref, h0_ref, h1_ref, dv_ref, b_ref, o_ref):
    dv = dv_ref[...]
    dv2 = dv * dv
    b = b_ref[...]
    m0 = dv * a0_ref[...] + dv2 * h0_ref[...] + b[:, :HD]
    m1 = dv * a1_ref[...] + dv2 * h1_ref[...] + b[:, HD:]
    o_ref[:, :HD] = jnp.maximum(m0, 0.0)
    o_ref[:, HD:] = jnp.maximum(m1, 0.0)


def _epilogue(a0, a1, h0, h1, dinv_col, b_row):
    return pl.pallas_call(
        _epi_body,
        grid=(N // _MMR,),
        in_specs=[pl.BlockSpec((_MMR, HD), lambda i: (i, 0)),
                  pl.BlockSpec((_MMR, HD), lambda i: (i, 0)),
                  pl.BlockSpec((_MMR, HD), lambda i: (i, 0)),
                  pl.BlockSpec((_MMR, HD), lambda i: (i, 0)),
                  pl.BlockSpec((_MMR, 1), lambda i: (i, 0)),
                  pl.BlockSpec((1, D), lambda i: (0, 0))],
        out_specs=pl.BlockSpec((_MMR, D), lambda i: (i, 0)),
        out_shape=jax.ShapeDtypeStruct((N, D), jnp.float32),
    )(a0, a1, h0, h1, dinv_col, b_row)


def kernel(X, edge_index, edge_weight, W1, b1, W2, b2):
    src = edge_index[0]
    dst = edge_index[1]
    pad_i = jnp.zeros((E_PAD - E,), jnp.int32)
    srcp = jnp.concatenate([src, pad_i])
    dstp = jnp.concatenate([dst, pad_i])
    ewp = jnp.concatenate([edge_weight, jnp.zeros((E_PAD - E,), jnp.float32)])

    dinv1d, sep = _prep(srcp, dstp, ewp)
    dinv_col = dinv1d[:N].reshape(N, 1)
    b1r = b1.reshape(1, D)
    b2r = b2.reshape(1, D)
    src3 = srcp.reshape(NS, NCH, ECH)
    dst3 = dstp.reshape(NS, NCH, ECH)
    se3 = sep.reshape(NS, NCH, ECH)

    zrows = jnp.zeros((N, HD), jnp.float32)

    def as_i32(hb):
        return lax.bitcast_convert_type(
            hb.reshape(N, HD // 2, 2), jnp.int32)

    h1a, h1b, h1ab, h1bb = _matmul_split(X, W1)
    a1a, a1b = _agg(as_i32(h1ab), as_i32(h1bb), src3, dst3, se3, zrows)
    out1 = _epilogue(a1a, a1b, h1a, h1b, dinv_col, b1r)

    h2a, h2b, h2ab, h2bb = _matmul_split(out1, W2)
    a2a, a2b = _agg(as_i32(h2ab), as_i32(h2bb), src3, dst3, se3, zrows)
    return _epilogue(a2a, a2b, h2a, h2b, dinv_col, b2r)


# speedup vs baseline: 1.1579x; 1.0390x over previous
import functools

import jax
import jax.numpy as jnp
from jax import lax
from jax.experimental import pallas as pl
from jax.experimental.pallas import tpu as pltpu
from jax.experimental.pallas import tpu_sc as plsc

N = 10000
E = 160000
D = 256
HD = 128
NC = 2
NS = 16
L = 16
E_PAD = 163840
EPT = E_PAD // NS
ECH = 128
NCH = EPT // ECH
DEG_CH = 1024
ESE = E_PAD // (NC * NS)
DRN = 632
DRL = 520
ZRN = 640
ZRL = 400

_vsm = plsc.VectorSubcoreMesh(
    core_axis_name="c", subcore_axis_name="s", num_cores=NC, num_subcores=NS)


def _nrsqrt(x):
    xi = plsc.bitcast(x, jnp.int32)
    y = plsc.bitcast(jnp.int32(0x5F3759DF) - (xi >> 1), jnp.float32)
    for _ in range(3):
        y = y * (1.5 - 0.5 * x * y * y)
    return y


NP = 10240
NPT = NP // NS


@functools.partial(
    pl.kernel,
    out_type=(jax.ShapeDtypeStruct((NP,), jnp.float32),
              jax.ShapeDtypeStruct((E_PAD,), jnp.float32)),
    mesh=_vsm,
    scratch_types=[
        pltpu.VMEM((NP,), jnp.float32),
        pltpu.VMEM((DEG_CH,), jnp.int32),
        pltpu.VMEM((DEG_CH,), jnp.float32),
        pltpu.VMEM((DEG_CH,), jnp.float32),
        pltpu.VMEM((NS, NPT), jnp.float32),
        pltpu.VMEM((NPT,), jnp.float32),
        pltpu.VMEM((NP,), jnp.float32),
        pltpu.VMEM_SHARED((NS, NP), jnp.float32),
        pltpu.VMEM_SHARED((NP,), jnp.float32),
    ],
    compiler_params=pltpu.CompilerParams(needs_layout_passes=False),
)
def _prep(src_hbm, dst_hbm, ew_hbm, dinv_hbm, se_hbm,
          deg_loc, ebuf_i, ebuf_f, se_buf, d2buf, dinv_loc, dinv_all,
          deg_sh, dinv_sh):
    c = lax.axis_index("c")
    s = lax.axis_index("s")
    zeros = jnp.zeros((L,), jnp.float32)

    def z_body(i, _):
        deg_loc[pl.ds(i * L, L)] = zeros
        return 0
    lax.fori_loop(0, NP // L, z_body, 0)

    def deg_chunk(ci, _):
        base = s * EPT + ci * DEG_CH
        pltpu.sync_copy(dst_hbm.at[pl.ds(base, DEG_CH)], ebuf_i)
        pltpu.sync_copy(ew_hbm.at[pl.ds(base, DEG_CH)], ebuf_f)

        def inner(i, _):
            dv = ebuf_i[pl.ds(i * L, L)]
            wv = ebuf_f[pl.ds(i * L, L)]
            plsc.addupdate_scatter(deg_loc, [dv], wv)
            return 0
        lax.fori_loop(0, DEG_CH // L, inner, 0)
        return 0
    lax.fori_loop(0, EPT // DEG_CH, deg_chunk, 0)

    pltpu.sync_copy(deg_loc, deg_sh.at[s])
    plsc.subcore_barrier()
    pltpu.sync_copy(deg_sh.at[:, pl.ds(s * NPT, NPT)], d2buf)

    def red_body(i, _):
        acc = d2buf[0, pl.ds(i * L, L)]
        for r in range(1, NS):
            acc = acc + d2buf[r, pl.ds(i * L, L)]
        dinv_loc[pl.ds(i * L, L)] = _nrsqrt(acc + 1.0)
        return 0
    lax.fori_loop(0, NPT // L, red_body, 0)
    pltpu.sync_copy(dinv_loc, dinv_sh.at[pl.ds(s * NPT, NPT)])

    @pl.when(c == 0)
    def _():
        pltpu.sync_copy(dinv_loc, dinv_hbm.at[pl.ds(s * NPT, NPT)])
    plsc.subcore_barrier()

    pltpu.sync_copy(dinv_sh, dinv_all)
    w = c * NS + s

    def se_chunk(ci, _):
        base = w * ESE + ci * DEG_CH
        pltpu.sync_copy(src_hbm.at[pl.ds(base, DEG_CH)], ebuf_i)
        pltpu.sync_copy(ew_hbm.at[pl.ds(base, DEG_CH)], ebuf_f)

        def inner(i, _):
            sv = ebuf_i[pl.ds(i * L, L)]
            dvv = plsc.load_gather(dinv_all, [sv])
            se_buf[pl.ds(i * L, L)] = ebuf_f[pl.ds(i * L, L)] * dvv
            return 0
        lax.fori_loop(0, DEG_CH // L, inner, 0)
        pltpu.sync_copy(se_buf, se_hbm.at[pl.ds(base, DEG_CH)])
        return 0
    lax.fori_loop(0, ESE // DEG_CH, se_chunk, 0)


@functools.partial(
    pl.kernel,
    out_type=(jax.ShapeDtypeStruct((N, HD), jnp.float32),
              jax.ShapeDtypeStruct((N, HD), jnp.float32)),
    mesh=_vsm,
    scratch_types=[
        pltpu.VMEM((2, ECH, HD // 2), jnp.int32),
        pltpu.VMEM((2, ECH, HD), jnp.float32),
        pltpu.VMEM((3, ECH), jnp.int32),
        pltpu.VMEM((3, ECH), jnp.int32),
        pltpu.VMEM((3, ECH), jnp.float32),
        pltpu.VMEM((ECH,), jnp.float32),
        pltpu.SemaphoreType.DMA,
        pltpu.SemaphoreType.DMA,
        pltpu.SemaphoreType.DMA,
        pltpu.VMEM_SHARED((N, HD), jnp.float32),
    ],
    compiler_params=pltpu.CompilerParams(
        needs_layout_passes=False, use_tc_tiling_on_sc=False),
)
def _agg(h0, h1, src_hbm, dst_hbm, se_hbm, z_hbm, o0, o1,
         gbuf3, msg2, src3b, dst3b, se3b, se_buf, gsem, ssem, isem, acc_sh):
    c = lax.axis_index("c")
    s = lax.axis_index("s")

    @pl.when(s < NS - 1)
    def _():
        pltpu.sync_copy(z_hbm.at[pl.ds(s * DRN, DRN)],
                        acc_sh.at[pl.ds(s * DRN, DRN)])

    @pl.when(s == NS - 1)
    def _():
        pltpu.sync_copy(z_hbm.at[pl.ds((NS - 1) * DRN, DRL)],
                        acc_sh.at[pl.ds((NS - 1) * DRN, DRL)])
    plsc.subcore_barrier()

    def run_half(h_hbm):
        def start_idx(j):
            r = lax.rem(j, 3)
            pltpu.async_copy(src_hbm.at[s, j], src3b.at[r], isem)
            pltpu.async_copy(dst_hbm.at[s, j], dst3b.at[r], isem)
            pltpu.async_copy(se_hbm.at[s, j], se3b.at[r], isem)

        def drain_idx():
            pltpu.make_async_copy(src_hbm.at[s, 0], src3b.at[0], isem).wait()
            pltpu.make_async_copy(dst_hbm.at[s, 0], dst3b.at[0], isem).wait()
            pltpu.make_async_copy(se_hbm.at[s, 0], se3b.at[0], isem).wait()

        def drain_gather():
            pltpu.make_async_copy(
                h_hbm.at[pl.ds(0, ECH)], gbuf3.at[0], gsem).wait()

        def drain_scatter():
            pltpu.make_async_copy(
                z_hbm.at[pl.ds(0, ECH)], msg2.at[0], ssem).wait()

        start_idx(0)
        start_idx(1)
        drain_idx()
        pltpu.async_copy(h_hbm.at[src3b.at[0]], gbuf3.at[0], gsem)

        def chunk(j, _):
            jb = lax.rem(j, 2)
            nb = lax.rem(j + 1, 2)
            jr = lax.rem(j, 3)

            @pl.when(j + 1 < NCH)
            def _():
                drain_idx()

            @pl.when(j + 2 < NCH)
            def _():
                start_idx(j + 2)

            @pl.when(j + 1 < NCH)
            def _():
                pltpu.async_copy(
                    h_hbm.at[src3b.at[lax.rem(j + 1, 3)]], gbuf3.at[nb], gsem)

            drain_gather()

            @pl.when(j >= 2)
            def _():
                drain_scatter()

            for k in range(ECH // L):
                se_buf[pl.ds(k * L, L)] = se3b[jr, pl.ds(k * L, L)]

            jbs = jnp.full((L,), jb, jnp.int32)
            evens = lax.iota(jnp.int32, L) * 2

            def edge(e, _):
                sv = plsc.load_gather(se_buf, [jnp.full((L,), e, jnp.int32)])
                es = jnp.full((L,), e, jnp.int32)
                for k in range(HD // 32):
                    w16 = gbuf3[jb, e, pl.ds(k * L, L)]
                    hb = plsc.bitcast(w16, jnp.bfloat16)
                    a, b = plsc.unpack(hb, format=plsc.PackFormat.INTERLEAVED)
                    plsc.store_scatter(msg2, [jbs, es, evens + (k * 32)],
                                       a * sv)
                    plsc.store_scatter(msg2, [jbs, es, evens + (k * 32 + 1)],
                                       b * sv)
                return 0
            lax.fori_loop(0, ECH, edge, 0)
            pltpu.async_copy(msg2.at[jb], acc_sh.at[dst3b.at[jr]],
                             ssem, add=True)
            return 0
        lax.fori_loop(0, NCH, chunk, 0)
        drain_scatter()
        drain_scatter()

    @pl.when(c == 0)
    def _():
        run_half(h0)

    @pl.when(c == 1)
    def _():
        run_half(h1)

    plsc.subcore_barrier()

    def drain(o_hbm):
        @pl.when(s < NS - 1)
        def _():
            pltpu.sync_copy(acc_sh.at[pl.ds(s * DRN, DRN)],
                            o_hbm.at[pl.ds(s * DRN, DRN)])

        @pl.when(s == NS - 1)
        def _():
            pltpu.sync_copy(acc_sh.at[pl.ds((NS - 1) * DRN, DRL)],
                            o_hbm.at[pl.ds((NS - 1) * DRN, DRL)])

    @pl.when(c == 0)
    def _():
        drain(o0)

    @pl.when(c == 1)
    def _():
        drain(o1)


def _mm_body(x_ref, w_ref, o0_ref, o1_ref, b0_ref, b1_ref):
    h = jnp.dot(x_ref[...], w_ref[...], preferred_element_type=jnp.float32,
                precision=lax.Precision.HIGHEST)
    h0 = h[:, :HD]
    h1 = h[:, HD:]
    o0_ref[...] = h0
    o1_ref[...] = h1
    b0_ref[...] = h0.astype(jnp.bfloat16)
    b1_ref[...] = h1.astype(jnp.bfloat16)


_MMR = 2000


def _matmul_split(x, w):
    return pl.pallas_call(
        _mm_body,
        grid=(N // _MMR,),
        in_specs=[pl.BlockSpec((_MMR, D), lambda i: (i, 0)),
                  pl.BlockSpec((D, D), lambda i: (0, 0))],
        out_specs=[pl.BlockSpec((_MMR, HD), lambda i: (i, 0)),
                   pl.BlockSpec((_MMR, HD), lambda i: (i, 0)),
                   pl.BlockSpec((_MMR, HD), lambda i: (i, 0)),
                   pl.BlockSpec((_MMR, HD), lambda i: (i, 0))],
        out_shape=[jax.ShapeDtypeStruct((N, HD), jnp.float32),
                   jax.ShapeDtypeStruct((N, HD), jnp.float32),
                   jax.ShapeDtypeStruct((N, HD), jnp.bfloat16),
                   jax.ShapeDtypeStruct((N, HD), jnp.bfloat16)],
    )(x, w)


def _epi_body(a0_ref, a1_ref, h0_ref, h1_ref, dv_ref, b_ref, o_ref):
    dv = dv_ref[...]
    dv2 = dv * dv
    b = b_ref[...]
    m0 = dv * a0_ref[...] + dv2 * h0_ref[...] + b[:, :HD]
    m1 = dv * a1_ref[...] + dv2 * h1_ref[...] + b[:, HD:]
    o_ref[:, :HD] = jnp.maximum(m0, 0.0)
    o_ref[:, HD:] = jnp.maximum(m1, 0.0)


def _epilogue(a0, a1, h0, h1, dinv_col, b_row):
    return pl.pallas_call(
        _epi_body,
        grid=(N // _MMR,),
        in_specs=[pl.BlockSpec((_MMR, HD), lambda i: (i, 0)),
                  pl.BlockSpec((_MMR, HD), lambda i: (i, 0)),
                  pl.BlockSpec((_MMR, HD), lambda i: (i, 0)),
                  pl.BlockSpec((_MMR, HD), lambda i: (i, 0)),
                  pl.BlockSpec((_MMR, 1), lambda i: (i, 0)),
                  pl.BlockSpec((1, D), lambda i: (0, 0))],
        out_specs=pl.BlockSpec((_MMR, D), lambda i: (i, 0)),
        out_shape=jax.ShapeDtypeStruct((N, D), jnp.float32),
    )(a0, a1, h0, h1, dinv_col, b_row)


def _emm_body(a0_ref, a1_ref, h0_ref, h1_ref, dv_ref, b_ref, w_ref,
              o0_ref, o1_ref, b0_ref, b1_ref):
    dv = dv_ref[...]
    dv2 = dv * dv
    b = b_ref[...]
    t0 = jnp.maximum(dv * a0_ref[...] + dv2 * h0_ref[...] + b[:, :HD], 0.0)
    t1 = jnp.maximum(dv * a1_ref[...] + dv2 * h1_ref[...] + b[:, HD:], 0.0)
    t = jnp.concatenate([t0, t1], axis=1)
    h = jnp.dot(t, w_ref[...], preferred_element_type=jnp.float32,
                precision=lax.Precision.HIGHEST)
    h0 = h[:, :HD]
    h1 = h[:, HD:]
    o0_ref[...] = h0
    o1_ref[...] = h1
    b0_ref[...] = h0.astype(jnp.bfloat16)
    b1_ref[...] = h1.astype(jnp.bfloat16)


def _epi_matmul(a0, a1, h0, h1, dinv_col, b_row, w):
    return pl.pallas_call(
        _emm_body,
        grid=(N // _MMR,),
        in_specs=[pl.BlockSpec((_MMR, HD), lambda i: (i, 0)),
                  pl.BlockSpec((_MMR, HD), lambda i: (i, 0)),
                  pl.BlockSpec((_MMR, HD), lambda i: (i, 0)),
                  pl.BlockSpec((_MMR, HD), lambda i: (i, 0)),
                  pl.BlockSpec((_MMR, 1), lambda i: (i, 0)),
                  pl.BlockSpec((1, D), lambda i: (0, 0)),
                  pl.BlockSpec((D, D), lambda i: (0, 0))],
        out_specs=[pl.BlockSpec((_MMR, HD), lambda i: (i, 0)),
                   pl.BlockSpec((_MMR, HD), lambda i: (i, 0)),
                   pl.BlockSpec((_MMR, HD), lambda i: (i, 0)),
                   pl.BlockSpec((_MMR, HD), lambda i: (i, 0))],
        out_shape=[jax.ShapeDtypeStruct((N, HD), jnp.float32),
                   jax.ShapeDtypeStruct((N, HD), jnp.float32),
                   jax.ShapeDtypeStruct((N, HD), jnp.bfloat16),
                   jax.ShapeDtypeStruct((N, HD), jnp.bfloat16)],
    )(a0, a1, h0, h1, dinv_col, b_row, w)


def kernel(X, edge_index, edge_weight, W1, b1, W2, b2):
    src = edge_index[0]
    dst = edge_index[1]
    pad_i = jnp.zeros((E_PAD - E,), jnp.int32)
    srcp = jnp.concatenate([src, pad_i])
    dstp = jnp.concatenate([dst, pad_i])
    ewp = jnp.concatenate([edge_weight, jnp.zeros((E_PAD - E,), jnp.float32)])

    dinv1d, sep = _prep(srcp, dstp, ewp)
    dinv_col = dinv1d[:N].reshape(N, 1)
    b1r = b1.reshape(1, D)
    b2r = b2.reshape(1, D)
    src3 = srcp.reshape(NS, NCH, ECH)
    dst3 = dstp.reshape(NS, NCH, ECH)
    se3 = sep.reshape(NS, NCH, ECH)

    zrows = jnp.zeros((N, HD), jnp.float32)

    def as_i32(hb):
        return lax.bitcast_convert_type(
            hb.reshape(N, HD // 2, 2), jnp.int32)

    h1a, h1b, h1ab, h1bb = _matmul_split(X, W1)
    a1a, a1b = _agg(as_i32(h1ab), as_i32(h1bb), src3, dst3, se3, zrows)

    h2a, h2b, h2ab, h2bb = _epi_matmul(a1a, a1b, h1a, h1b, dinv_col, b1r, W2)
    a2a, a2b = _agg(as_i32(h2ab), as_i32(h2bb), src3, dst3, se3, zrows)
    return _epilogue(a2a, a2b, h2a, h2b, dinv_col, b2r)
